# baseline (speedup 1.0000x reference)
import functools

import jax
import jax.numpy as jnp
import numpy as np
from jax import lax
from jax.experimental import pallas as pl
from jax.experimental.pallas import tpu as pltpu
from jax.experimental.pallas import tpu_sc as plsc

E = 8
D = 1024
F_MOE = 1024
F_SHARED = 2048
N_TOK = 4096
P = 2 * N_TOK
TB = 256
NB = N_TOK // TB
TMG = 128
RMAX = P + E * TMG
NBG = RMAX // TMG
COLS = 16
ROWS = P // COLS

SC_NC = 2
SC_NS = 16
NW = SC_NC * SC_NS
TPW = N_TOK // NW
CH = 64
NCH = TPW // CH


def _dot_t(a, b):
    return lax.dot_general(a, b, (((1,), (1,)), ((), ())),
                           preferred_element_type=jnp.float32)


def _shared_router_body(x_ref, wg_ref, wsgu_ref, wsdn_ref, wsg_ref,
                        shared_ref, wts_ref, eidx_ref):
    x = x_ref[...]
    xb = x.astype(jnp.bfloat16)
    sgu = _dot_t(xb, wsgu_ref[...])
    sg = sgu[:, :F_SHARED]
    su = sgu[:, F_SHARED:]
    sh = (sg * jax.nn.sigmoid(sg) * su).astype(jnp.bfloat16)
    sout = _dot_t(sh, wsdn_ref[...])
    gate = jax.nn.sigmoid(_dot_t(x, wsg_ref[...]))
    shared_ref[...] = gate * sout

    logits = _dot_t(x, wg_ref[...])
    iota = lax.broadcasted_iota(jnp.int32, logits.shape, 1)
    m1 = jnp.max(logits, axis=1, keepdims=True)
    a1 = jnp.min(jnp.where(logits == m1, iota, E + 1), axis=1, keepdims=True)
    l2 = jnp.where(iota == a1, -jnp.inf, logits)
    m2 = jnp.max(l2, axis=1, keepdims=True)
    a2 = jnp.min(jnp.where(l2 == m2, iota, E + 1), axis=1, keepdims=True)
    w1 = 1.0 / (1.0 + jnp.exp(m2 - m1))
    wts_ref[...] = jnp.concatenate([w1, 1.0 - w1], axis=1)
    eidx_ref[...] = jnp.concatenate([a1, a2], axis=1)


_shared_router_call = pl.pallas_call(
    _shared_router_body,
    grid=(NB,),
    in_specs=[
        pl.BlockSpec((TB, D), lambda b: (b, 0)),
        pl.BlockSpec((E, D), lambda b: (0, 0)),
        pl.BlockSpec((2 * F_SHARED, D), lambda b: (0, 0)),
        pl.BlockSpec((D, F_SHARED), lambda b: (0, 0)),
        pl.BlockSpec((1, D), lambda b: (0, 0)),
    ],
    out_specs=[
        pl.BlockSpec((TB, D), lambda b: (b, 0)),
        pl.BlockSpec((TB, 2), lambda b: (b, 0)),
        pl.BlockSpec((TB, 2), lambda b: (b, 0)),
    ],
    out_shape=[
        jax.ShapeDtypeStruct((N_TOK, D), jnp.float32),
        jax.ShapeDtypeStruct((N_TOK, 2), jnp.float32),
        jax.ShapeDtypeStruct((N_TOK, 2), jnp.int32),
    ],
)


def _sort_body(e_ref, lt_ref, cp_ref, dest_ref, bexp_ref):
    e = e_ref[...]
    lt = lt_ref[...]
    cp = cp_ref[...]

    dest = jnp.zeros((ROWS, COLS), jnp.float32)
    off = jnp.zeros((1, 1), jnp.float32)
    rb = (TMG * lax.broadcasted_iota(jnp.int32, (8, NBG), 1)).astype(jnp.float32)
    nleq = jnp.zeros((8, NBG), jnp.float32)
    for k in range(E):
        ohk = (e == k).astype(jnp.float32)
        within = lax.dot_general(lt, ohk, (((1,), (0,)), ((), ())),
                                 preferred_element_type=jnp.float32)
        s = within[ROWS - 1:ROWS, :]
        excl = lax.dot_general(s, cp, (((1,), (0,)), ((), ())),
                               preferred_element_type=jnp.float32)
        incl = within + excl
        dest = dest + ohk * (off + incl - 1.0)
        nleq = nleq + (off <= rb).astype(jnp.float32)
        tot = excl[0:1, COLS - 1:COLS] + s[0:1, COLS - 1:COLS]
        off = off + jnp.ceil(tot * (1.0 / TMG)) * TMG
    dest_ref[...] = dest.astype(jnp.int32)
    bexp_ref[...] = (nleq - 1.0).astype(jnp.int32)


_sort_call = pl.pallas_call(
    _sort_body,
    grid=(1,),
    in_specs=[
        pl.BlockSpec((ROWS, COLS), lambda i: (0, 0)),
        pl.BlockSpec((ROWS, ROWS), lambda i: (0, 0)),
        pl.BlockSpec((COLS, COLS), lambda i: (0, 0)),
    ],
    out_specs=[
        pl.BlockSpec((ROWS, COLS), lambda i: (0, 0)),
        pl.BlockSpec((8, NBG), lambda i: (0, 0)),
    ],
    out_shape=[
        jax.ShapeDtypeStruct((ROWS, COLS), jnp.int32),
        jax.ShapeDtypeStruct((8, NBG), jnp.int32),
    ],
)

_LT = np.tril(np.ones((ROWS, ROWS), np.float32))
_CP = np.tril(np.ones((COLS, COLS), np.float32), -1).T


@functools.cache
def _sc_calls():
    mesh = plsc.VectorSubcoreMesh(core_axis_name="c", subcore_axis_name="s",
                                  num_cores=SC_NC, num_subcores=SC_NS)

    @functools.partial(
        pl.kernel,
        out_type=jax.ShapeDtypeStruct((RMAX, D), jnp.float32),
        mesh=mesh,
        scratch_types=[
            pltpu.VMEM((CH,), jnp.int32),
            pltpu.VMEM((CH,), jnp.int32),
            pltpu.VMEM((CH, D), jnp.float32),
            pltpu.SemaphoreType.DMA,
        ],
    )
    def _sc_scatter(x_hbm, d0_hbm, d1_hbm, xs_hbm, idx0_v, idx1_v, rows_v, sem):
        wid = lax.axis_index("s") * SC_NC + lax.axis_index("c")
        for c in range(NCH):
            base = wid * TPW + c * CH
            pltpu.sync_copy(d0_hbm.at[pl.ds(base, CH)], idx0_v)
            pltpu.sync_copy(d1_hbm.at[pl.ds(base, CH)], idx1_v)
            pltpu.sync_copy(x_hbm.at[pl.ds(base, CH)], rows_v)
            pltpu.async_copy(rows_v, xs_hbm.at[idx0_v], sem).wait()
            pltpu.async_copy(rows_v, xs_hbm.at[idx1_v], sem).wait()

    @functools.partial(
        pl.kernel,
        out_type=[
            jax.ShapeDtypeStruct((N_TOK, D), jnp.float32),
            jax.ShapeDtypeStruct((N_TOK, D), jnp.float32),
        ],
        mesh=mesh,
        scratch_types=[
            pltpu.VMEM((CH,), jnp.int32),
            pltpu.VMEM((CH, D), jnp.float32),
            pltpu.SemaphoreType.DMA,
        ],
    )
    def _sc_gather(ys_hbm, d0_hbm, d1_hbm, y0_hbm, y1_hbm, idx_v, rows_v, sem):
        wid = lax.axis_index("s") * SC_NC + lax.axis_index("c")
        for c in range(NCH):
            base = wid * TPW + c * CH
            pltpu.sync_copy(d0_hbm.at[pl.ds(base, CH)], idx_v)
            pltpu.async_copy(ys_hbm.at[idx_v], rows_v, sem).wait()
            pltpu.sync_copy(rows_v, y0_hbm.at[pl.ds(base, CH)])
            pltpu.sync_copy(d1_hbm.at[pl.ds(base, CH)], idx_v)
            pltpu.async_copy(ys_hbm.at[idx_v], rows_v, sem).wait()
            pltpu.sync_copy(rows_v, y1_hbm.at[pl.ds(base, CH)])

    return _sc_scatter, _sc_gather


def _group_mlp_body(bexp_ref, xs_ref, wgu_ref, wdn_ref, ys_ref):
    x = xs_ref[...].astype(jnp.bfloat16)
    gu = _dot_t(x, wgu_ref[0])
    g = gu[:, :F_MOE]
    u = gu[:, F_MOE:]
    h = (g * jax.nn.sigmoid(g) * u).astype(jnp.bfloat16)
    ys_ref[...] = _dot_t(h, wdn_ref[0])


_group_mlp_call = pl.pallas_call(
    _group_mlp_body,
    grid_spec=pltpu.PrefetchScalarGridSpec(
        num_scalar_prefetch=1,
        grid=(NBG,),
        in_specs=[
            pl.BlockSpec((TMG, D), lambda b, s: (b, 0)),
            pl.BlockSpec((1, 2 * F_MOE, D), lambda b, s: (s[b], 0, 0)),
            pl.BlockSpec((1, D, F_MOE), lambda b, s: (s[b], 0, 0)),
        ],
        out_specs=pl.BlockSpec((TMG, D), lambda b, s: (b, 0)),
    ),
    out_shape=jax.ShapeDtypeStruct((RMAX, D), jnp.float32),
)


def _combine_body(shared_ref, wts_ref, y0_ref, y1_ref, out_ref):
    w = wts_ref[...]
    out_ref[...] = (shared_ref[...] + w[:, 0:1] * y0_ref[...]
                    + w[:, 1:2] * y1_ref[...])


_combine_call = pl.pallas_call(
    _combine_body,
    grid=(NB,),
    in_specs=[
        pl.BlockSpec((TB, D), lambda b: (b, 0)),
        pl.BlockSpec((TB, 2), lambda b: (b, 0)),
        pl.BlockSpec((TB, D), lambda b: (b, 0)),
        pl.BlockSpec((TB, D), lambda b: (b, 0)),
    ],
    out_specs=pl.BlockSpec((TB, D), lambda b: (b, 0)),
    out_shape=jax.ShapeDtypeStruct((N_TOK, D), jnp.float32),
)


@jax.jit
def kernel(hidden_states, Wg, W_gu, W_dn, Ws_gu, Ws_dn, Wsg):
    bs, nt, d = hidden_states.shape
    x = hidden_states.reshape(-1, d)

    shared, wts, eidx = _shared_router_call(
        x, Wg, Ws_gu.astype(jnp.bfloat16), Ws_dn.astype(jnp.bfloat16), Wsg)

    e_cols = eidx.reshape(P).reshape(COLS, ROWS).T
    dest_cols, bexp = _sort_call(e_cols, _LT, _CP)
    dest = dest_cols.T.reshape(P).reshape(N_TOK, 2)
    d0 = dest[:, 0]
    d1 = dest[:, 1]

    sc_scatter, sc_gather = _sc_calls()
    xs = sc_scatter(x, d0, d1)
    ys = _group_mlp_call(bexp[0], xs, W_gu.astype(jnp.bfloat16),
                         W_dn.astype(jnp.bfloat16))
    y0, y1 = sc_gather(ys, d0, d1)

    out = _combine_call(shared, wts, y0, y1)
    return out.reshape(bs, nt, d)


# speedup vs baseline: 1.1055x; 1.1055x over previous
import functools

import jax
import jax.numpy as jnp
import numpy as np
from jax import lax
from jax.experimental import pallas as pl
from jax.experimental.pallas import tpu as pltpu
from jax.experimental.pallas import tpu_sc as plsc

E = 8
D = 1024
F_MOE = 1024
F_SHARED = 2048
N_TOK = 4096
P = 2 * N_TOK
TB = 256
NB = N_TOK // TB
TMG = 128
RMAX = P + E * TMG
NBG = RMAX // TMG
COLS = 16
ROWS = P // COLS

SC_NC = 2
SC_NS = 16
NW = SC_NC * SC_NS
TPW = N_TOK // NW
CH = 64
NCH = TPW // CH


def _dot_t(a, b):
    return lax.dot_general(a, b, (((1,), (1,)), ((), ())),
                           preferred_element_type=jnp.float32)


def _shared_router_body(x_ref, wg_ref, wsgu_ref, wsdn_ref, wsg_ref,
                        shared_ref, wts_ref, eidx_ref):
    x = x_ref[...]
    sgu = _dot_t(x, wsgu_ref[...])
    sg = sgu[:, :F_SHARED]
    su = sgu[:, F_SHARED:]
    sh = sg * jax.nn.sigmoid(sg) * su
    sout = _dot_t(sh, wsdn_ref[...])
    gate = jax.nn.sigmoid(_dot_t(x, wsg_ref[...]))
    shared_ref[...] = gate * sout

    logits = _dot_t(x, wg_ref[...])
    iota = lax.broadcasted_iota(jnp.int32, logits.shape, 1)
    m1 = jnp.max(logits, axis=1, keepdims=True)
    a1 = jnp.min(jnp.where(logits == m1, iota, E + 1), axis=1, keepdims=True)
    l2 = jnp.where(iota == a1, -jnp.inf, logits)
    m2 = jnp.max(l2, axis=1, keepdims=True)
    a2 = jnp.min(jnp.where(l2 == m2, iota, E + 1), axis=1, keepdims=True)
    w1 = 1.0 / (1.0 + jnp.exp(m2 - m1))
    wts_ref[...] = jnp.concatenate([w1, 1.0 - w1], axis=1)
    eidx_ref[...] = jnp.concatenate([a1, a2], axis=1)


_shared_router_call = pl.pallas_call(
    _shared_router_body,
    grid=(NB,),
    in_specs=[
        pl.BlockSpec((TB, D), lambda b: (b, 0)),
        pl.BlockSpec((E, D), lambda b: (0, 0)),
        pl.BlockSpec((2 * F_SHARED, D), lambda b: (0, 0)),
        pl.BlockSpec((D, F_SHARED), lambda b: (0, 0)),
        pl.BlockSpec((1, D), lambda b: (0, 0)),
    ],
    out_specs=[
        pl.BlockSpec((TB, D), lambda b: (b, 0)),
        pl.BlockSpec((TB, 2), lambda b: (b, 0)),
        pl.BlockSpec((TB, 2), lambda b: (b, 0)),
    ],
    out_shape=[
        jax.ShapeDtypeStruct((N_TOK, D), jnp.float32),
        jax.ShapeDtypeStruct((N_TOK, 2), jnp.float32),
        jax.ShapeDtypeStruct((N_TOK, 2), jnp.int32),
    ],
)


def _sort_body(e_ref, lt_ref, cp_ref, dest_ref, bexp_ref):
    e = e_ref[...]
    lt = lt_ref[...]
    cp = cp_ref[...]

    dest = jnp.zeros((ROWS, COLS), jnp.float32)
    off = jnp.zeros((1, 1), jnp.float32)
    rb = (TMG * lax.broadcasted_iota(jnp.int32, (8, NBG), 1)).astype(jnp.float32)
    nleq = jnp.zeros((8, NBG), jnp.float32)
    for k in range(E):
        ohk = (e == k).astype(jnp.float32)
        within = lax.dot_general(lt, ohk, (((1,), (0,)), ((), ())),
                                 preferred_element_type=jnp.float32)
        s = within[ROWS - 1:ROWS, :]
        excl = lax.dot_general(s, cp, (((1,), (0,)), ((), ())),
                               preferred_element_type=jnp.float32)
        incl = within + excl
        dest = dest + ohk * (off + incl - 1.0)
        nleq = nleq + (off <= rb).astype(jnp.float32)
        tot = excl[0:1, COLS - 1:COLS] + s[0:1, COLS - 1:COLS]
        off = off + jnp.ceil(tot * (1.0 / TMG)) * TMG
    dest_ref[...] = dest.astype(jnp.int32)
    bexp_ref[...] = (nleq - 1.0).astype(jnp.int32)


_sort_call = pl.pallas_call(
    _sort_body,
    grid=(1,),
    in_specs=[
        pl.BlockSpec((ROWS, COLS), lambda i: (0, 0)),
        pl.BlockSpec((ROWS, ROWS), lambda i: (0, 0)),
        pl.BlockSpec((COLS, COLS), lambda i: (0, 0)),
    ],
    out_specs=[
        pl.BlockSpec((ROWS, COLS), lambda i: (0, 0)),
        pl.BlockSpec((8, NBG), lambda i: (0, 0)),
    ],
    out_shape=[
        jax.ShapeDtypeStruct((ROWS, COLS), jnp.int32),
        jax.ShapeDtypeStruct((8, NBG), jnp.int32),
    ],
)

_LT = np.tril(np.ones((ROWS, ROWS), np.float32))
_CP = np.tril(np.ones((COLS, COLS), np.float32), -1).T


@functools.cache
def _sc_calls():
    mesh = plsc.VectorSubcoreMesh(core_axis_name="c", subcore_axis_name="s",
                                  num_cores=SC_NC, num_subcores=SC_NS)

    @functools.partial(
        pl.kernel,
        out_type=jax.ShapeDtypeStruct((RMAX, D), jnp.float32),
        mesh=mesh,
        scratch_types=[
            pltpu.VMEM((CH,), jnp.int32),
            pltpu.VMEM((CH,), jnp.int32),
            pltpu.VMEM((CH, D), jnp.float32),
            pltpu.SemaphoreType.DMA,
        ],
    )
    def _sc_scatter(x_hbm, d0_hbm, d1_hbm, xs_hbm, idx0_v, idx1_v, rows_v, sem):
        wid = lax.axis_index("s") * SC_NC + lax.axis_index("c")
        for c in range(NCH):
            base = wid * TPW + c * CH
            pltpu.sync_copy(d0_hbm.at[pl.ds(base, CH)], idx0_v)
            pltpu.sync_copy(d1_hbm.at[pl.ds(base, CH)], idx1_v)
            pltpu.sync_copy(x_hbm.at[pl.ds(base, CH)], rows_v)
            pltpu.async_copy(rows_v, xs_hbm.at[idx0_v], sem).wait()
            pltpu.async_copy(rows_v, xs_hbm.at[idx1_v], sem).wait()

    @functools.partial(
        pl.kernel,
        out_type=[
            jax.ShapeDtypeStruct((N_TOK, D), jnp.float32),
            jax.ShapeDtypeStruct((N_TOK, D), jnp.float32),
        ],
        mesh=mesh,
        scratch_types=[
            pltpu.VMEM((CH,), jnp.int32),
            pltpu.VMEM((CH, D), jnp.float32),
            pltpu.SemaphoreType.DMA,
        ],
    )
    def _sc_gather(ys_hbm, d0_hbm, d1_hbm, y0_hbm, y1_hbm, idx_v, rows_v, sem):
        wid = lax.axis_index("s") * SC_NC + lax.axis_index("c")
        for c in range(NCH):
            base = wid * TPW + c * CH
            pltpu.sync_copy(d0_hbm.at[pl.ds(base, CH)], idx_v)
            pltpu.async_copy(ys_hbm.at[idx_v], rows_v, sem).wait()
            pltpu.sync_copy(rows_v, y0_hbm.at[pl.ds(base, CH)])
            pltpu.sync_copy(d1_hbm.at[pl.ds(base, CH)], idx_v)
            pltpu.async_copy(ys_hbm.at[idx_v], rows_v, sem).wait()
            pltpu.sync_copy(rows_v, y1_hbm.at[pl.ds(base, CH)])

    return _sc_scatter, _sc_gather


def _group_mlp_body(bexp_ref, xs_ref, wgu_ref, wdn_ref, ys_ref):
    x = xs_ref[...]
    gu = _dot_t(x, wgu_ref[0])
    g = gu[:, :F_MOE]
    u = gu[:, F_MOE:]
    h = g * jax.nn.sigmoid(g) * u
    ys_ref[...] = _dot_t(h, wdn_ref[0])


_group_mlp_call = pl.pallas_call(
    _group_mlp_body,
    grid_spec=pltpu.PrefetchScalarGridSpec(
        num_scalar_prefetch=1,
        grid=(NBG,),
        in_specs=[
            pl.BlockSpec((TMG, D), lambda b, s: (b, 0)),
            pl.BlockSpec((1, 2 * F_MOE, D), lambda b, s: (s[b], 0, 0)),
            pl.BlockSpec((1, D, F_MOE), lambda b, s: (s[b], 0, 0)),
        ],
        out_specs=pl.BlockSpec((TMG, D), lambda b, s: (b, 0)),
    ),
    out_shape=jax.ShapeDtypeStruct((RMAX, D), jnp.float32),
)


def _combine_body(shared_ref, wts_ref, y0_ref, y1_ref, out_ref):
    w = wts_ref[...]
    out_ref[...] = (shared_ref[...] + w[:, 0:1] * y0_ref[...]
                    + w[:, 1:2] * y1_ref[...])


_combine_call = pl.pallas_call(
    _combine_body,
    grid=(NB,),
    in_specs=[
        pl.BlockSpec((TB, D), lambda b: (b, 0)),
        pl.BlockSpec((TB, 2), lambda b: (b, 0)),
        pl.BlockSpec((TB, D), lambda b: (b, 0)),
        pl.BlockSpec((TB, D), lambda b: (b, 0)),
    ],
    out_specs=pl.BlockSpec((TB, D), lambda b: (b, 0)),
    out_shape=jax.ShapeDtypeStruct((N_TOK, D), jnp.float32),
)


@jax.jit
def kernel(hidden_states, Wg, W_gu, W_dn, Ws_gu, Ws_dn, Wsg):
    bs, nt, d = hidden_states.shape
    x = hidden_states.reshape(-1, d)

    shared, wts, eidx = _shared_router_call(x, Wg, Ws_gu, Ws_dn, Wsg)

    e_cols = eidx.reshape(P).reshape(COLS, ROWS).T
    dest_cols, bexp = _sort_call(e_cols, _LT, _CP)
    dest = dest_cols.T.reshape(P).reshape(N_TOK, 2)
    d0 = dest[:, 0]
    d1 = dest[:, 1]

    sc_scatter, sc_gather = _sc_calls()
    xs = sc_scatter(x, d0, d1)
    ys = _group_mlp_call(bexp[0], xs, W_gu, W_dn)
    y0, y1 = sc_gather(ys, d0, d1)

    out = _combine_call(shared, wts, y0, y1)
    return out.reshape(bs, nt, d)


# speedup vs baseline: 1.4070x; 1.2728x over previous
import functools

import jax
import jax.numpy as jnp
import numpy as np
from jax import lax
from jax.experimental import pallas as pl
from jax.experimental.pallas import tpu as pltpu
from jax.experimental.pallas import tpu_sc as plsc

E = 8
D = 1024
F_MOE = 1024
F_SHARED = 2048
N_TOK = 4096
P = 2 * N_TOK
TB = 256
NB = N_TOK // TB
TMG = 512
RMAX = P + E * TMG
NBG = RMAX // TMG
COLS = 16
ROWS = P // COLS

SC_NC = 2
SC_NS = 16
NW = SC_NC * SC_NS
TPW = N_TOK // NW
CH = 64
NCH = TPW // CH


def _dot_t(a, b):
    return lax.dot_general(a, b, (((1,), (1,)), ((), ())),
                           preferred_element_type=jnp.float32)


def _shared_router_body(x_ref, wg_ref, wsgu_ref, wsdn_ref, wsg_ref,
                        shared_ref, wts_ref, eidx_ref):
    x = x_ref[...]
    sgu = _dot_t(x, wsgu_ref[...])
    sg = sgu[:, :F_SHARED]
    su = sgu[:, F_SHARED:]
    sh = sg * jax.nn.sigmoid(sg) * su
    sout = _dot_t(sh, wsdn_ref[...])
    gate = jax.nn.sigmoid(_dot_t(x, wsg_ref[...]))
    shared_ref[...] = gate * sout

    logits = _dot_t(x, wg_ref[...])
    iota = lax.broadcasted_iota(jnp.int32, logits.shape, 1)
    m1 = jnp.max(logits, axis=1, keepdims=True)
    a1 = jnp.min(jnp.where(logits == m1, iota, E + 1), axis=1, keepdims=True)
    l2 = jnp.where(iota == a1, -jnp.inf, logits)
    m2 = jnp.max(l2, axis=1, keepdims=True)
    a2 = jnp.min(jnp.where(l2 == m2, iota, E + 1), axis=1, keepdims=True)
    w1 = 1.0 / (1.0 + jnp.exp(m2 - m1))
    wts_ref[...] = jnp.concatenate([w1, 1.0 - w1], axis=1)
    eidx_ref[...] = jnp.concatenate([a1, a2], axis=1)


_shared_router_call = pl.pallas_call(
    _shared_router_body,
    grid=(NB,),
    in_specs=[
        pl.BlockSpec((TB, D), lambda b: (b, 0)),
        pl.BlockSpec((E, D), lambda b: (0, 0)),
        pl.BlockSpec((2 * F_SHARED, D), lambda b: (0, 0)),
        pl.BlockSpec((D, F_SHARED), lambda b: (0, 0)),
        pl.BlockSpec((1, D), lambda b: (0, 0)),
    ],
    out_specs=[
        pl.BlockSpec((TB, D), lambda b: (b, 0)),
        pl.BlockSpec((TB, 2), lambda b: (b, 0)),
        pl.BlockSpec((TB, 2), lambda b: (b, 0)),
    ],
    out_shape=[
        jax.ShapeDtypeStruct((N_TOK, D), jnp.float32),
        jax.ShapeDtypeStruct((N_TOK, 2), jnp.float32),
        jax.ShapeDtypeStruct((N_TOK, 2), jnp.int32),
    ],
)


def _sort_body(e_ref, lt_ref, cp_ref, dest_ref, bexp_ref):
    e = e_ref[...]
    lt = lt_ref[...]
    cp = cp_ref[...]

    dest = jnp.zeros((ROWS, COLS), jnp.float32)
    off = jnp.zeros((1, 1), jnp.float32)
    rb = (TMG * lax.broadcasted_iota(jnp.int32, (8, NBG), 1)).astype(jnp.float32)
    nleq = jnp.zeros((8, NBG), jnp.float32)
    for k in range(E):
        ohk = (e == k).astype(jnp.float32)
        within = lax.dot_general(lt, ohk, (((1,), (0,)), ((), ())),
                                 preferred_element_type=jnp.float32)
        s = within[ROWS - 1:ROWS, :]
        excl = lax.dot_general(s, cp, (((1,), (0,)), ((), ())),
                               preferred_element_type=jnp.float32)
        incl = within + excl
        dest = dest + ohk * (off + incl - 1.0)
        nleq = nleq + (off <= rb).astype(jnp.float32)
        tot = excl[0:1, COLS - 1:COLS] + s[0:1, COLS - 1:COLS]
        off = off + jnp.ceil(tot * (1.0 / TMG)) * TMG
    dest_ref[...] = dest.astype(jnp.int32)
    bexp_ref[...] = (nleq - 1.0).astype(jnp.int32)


_sort_call = pl.pallas_call(
    _sort_body,
    grid=(1,),
    in_specs=[
        pl.BlockSpec((ROWS, COLS), lambda i: (0, 0)),
        pl.BlockSpec((ROWS, ROWS), lambda i: (0, 0)),
        pl.BlockSpec((COLS, COLS), lambda i: (0, 0)),
    ],
    out_specs=[
        pl.BlockSpec((ROWS, COLS), lambda i: (0, 0)),
        pl.BlockSpec((8, NBG), lambda i: (0, 0)),
    ],
    out_shape=[
        jax.ShapeDtypeStruct((ROWS, COLS), jnp.int32),
        jax.ShapeDtypeStruct((8, NBG), jnp.int32),
    ],
)

_LT = np.tril(np.ones((ROWS, ROWS), np.float32))
_CP = np.tril(np.ones((COLS, COLS), np.float32), -1).T


@functools.cache
def _sc_calls():
    mesh = plsc.VectorSubcoreMesh(core_axis_name="c", subcore_axis_name="s",
                                  num_cores=SC_NC, num_subcores=SC_NS)

    @functools.partial(
        pl.kernel,
        out_type=jax.ShapeDtypeStruct((RMAX, D), jnp.float32),
        mesh=mesh,
        scratch_types=[
            pltpu.VMEM((CH,), jnp.int32),
            pltpu.VMEM((CH,), jnp.int32),
            pltpu.VMEM((CH, D), jnp.float32),
            pltpu.SemaphoreType.DMA,
        ],
    )
    def _sc_scatter(x_hbm, d0_hbm, d1_hbm, xs_hbm, idx0_v, idx1_v, rows_v, sem):
        wid = lax.axis_index("s") * SC_NC + lax.axis_index("c")
        for c in range(NCH):
            base = wid * TPW + c * CH
            pltpu.sync_copy(d0_hbm.at[pl.ds(base, CH)], idx0_v)
            pltpu.sync_copy(d1_hbm.at[pl.ds(base, CH)], idx1_v)
            pltpu.sync_copy(x_hbm.at[pl.ds(base, CH)], rows_v)
            pltpu.async_copy(rows_v, xs_hbm.at[idx0_v], sem).wait()
            pltpu.async_copy(rows_v, xs_hbm.at[idx1_v], sem).wait()

    @functools.partial(
        pl.kernel,
        out_type=[
            jax.ShapeDtypeStruct((N_TOK, D), jnp.float32),
            jax.ShapeDtypeStruct((N_TOK, D), jnp.float32),
        ],
        mesh=mesh,
        scratch_types=[
            pltpu.VMEM((CH,), jnp.int32),
            pltpu.VMEM((CH, D), jnp.float32),
            pltpu.SemaphoreType.DMA,
        ],
    )
    def _sc_gather(ys_hbm, d0_hbm, d1_hbm, y0_hbm, y1_hbm, idx_v, rows_v, sem):
        wid = lax.axis_index("s") * SC_NC + lax.axis_index("c")
        for c in range(NCH):
            base = wid * TPW + c * CH
            pltpu.sync_copy(d0_hbm.at[pl.ds(base, CH)], idx_v)
            pltpu.async_copy(ys_hbm.at[idx_v], rows_v, sem).wait()
            pltpu.sync_copy(rows_v, y0_hbm.at[pl.ds(base, CH)])
            pltpu.sync_copy(d1_hbm.at[pl.ds(base, CH)], idx_v)
            pltpu.async_copy(ys_hbm.at[idx_v], rows_v, sem).wait()
            pltpu.sync_copy(rows_v, y1_hbm.at[pl.ds(base, CH)])

    return _sc_scatter, _sc_gather


def _group_mlp_body(bexp_ref, xs_ref, wgu_ref, wdn_ref, ys_ref):
    x = xs_ref[...]
    gu = _dot_t(x, wgu_ref[0])
    g = gu[:, :F_MOE]
    u = gu[:, F_MOE:]
    h = g * jax.nn.sigmoid(g) * u
    ys_ref[...] = _dot_t(h, wdn_ref[0])


_group_mlp_call = pl.pallas_call(
    _group_mlp_body,
    grid_spec=pltpu.PrefetchScalarGridSpec(
        num_scalar_prefetch=1,
        grid=(NBG,),
        in_specs=[
            pl.BlockSpec((TMG, D), lambda b, s: (b, 0)),
            pl.BlockSpec((1, 2 * F_MOE, D), lambda b, s: (s[b], 0, 0)),
            pl.BlockSpec((1, D, F_MOE), lambda b, s: (s[b], 0, 0)),
        ],
        out_specs=pl.BlockSpec((TMG, D), lambda b, s: (b, 0)),
    ),
    out_shape=jax.ShapeDtypeStruct((RMAX, D), jnp.float32),
)


def _combine_body(shared_ref, wts_ref, y0_ref, y1_ref, out_ref):
    w = wts_ref[...]
    out_ref[...] = (shared_ref[...] + w[:, 0:1] * y0_ref[...]
                    + w[:, 1:2] * y1_ref[...])


_combine_call = pl.pallas_call(
    _combine_body,
    grid=(NB,),
    in_specs=[
        pl.BlockSpec((TB, D), lambda b: (b, 0)),
        pl.BlockSpec((TB, 2), lambda b: (b, 0)),
        pl.BlockSpec((TB, D), lambda b: (b, 0)),
        pl.BlockSpec((TB, D), lambda b: (b, 0)),
    ],
    out_specs=pl.BlockSpec((TB, D), lambda b: (b, 0)),
    out_shape=jax.ShapeDtypeStruct((N_TOK, D), jnp.float32),
)


@jax.jit
def kernel(hidden_states, Wg, W_gu, W_dn, Ws_gu, Ws_dn, Wsg):
    bs, nt, d = hidden_states.shape
    x = hidden_states.reshape(-1, d)

    shared, wts, eidx = _shared_router_call(x, Wg, Ws_gu, Ws_dn, Wsg)

    e_cols = eidx.reshape(P).reshape(COLS, ROWS).T
    dest_cols, bexp = _sort_call(e_cols, _LT, _CP)
    dest = dest_cols.T.reshape(P).reshape(N_TOK, 2)
    d0 = dest[:, 0]
    d1 = dest[:, 1]

    sc_scatter, sc_gather = _sc_calls()
    xs = sc_scatter(x, d0, d1)
    ys = _group_mlp_call(bexp[0], xs, W_gu, W_dn)
    y0, y1 = sc_gather(ys, d0, d1)

    out = _combine_call(shared, wts, y0, y1)
    return out.reshape(bs, nt, d)


# speedup vs baseline: 1.4788x; 1.0510x over previous
import functools

import jax
import jax.numpy as jnp
import numpy as np
from jax import lax
from jax.experimental import pallas as pl
from jax.experimental.pallas import tpu as pltpu
from jax.experimental.pallas import tpu_sc as plsc

E = 8
D = 1024
F_MOE = 1024
F_SHARED = 2048
N_TOK = 4096
P = 2 * N_TOK
TB = 256
NB = N_TOK // TB
TMG = 256
RMAX = P + E * TMG
NBG = RMAX // TMG
COLS = 16
ROWS = P // COLS

SC_NC = 2
SC_NS = 16
NW = SC_NC * SC_NS
TPW = N_TOK // NW
CH = 64
NCH = TPW // CH


def _dot_t(a, b):
    return lax.dot_general(a, b, (((1,), (1,)), ((), ())),
                           preferred_element_type=jnp.float32)


def _router_body(x_ref, wg_ref, wts_ref, eidx_ref):
    logits = _dot_t(x_ref[...], wg_ref[...])
    iota = lax.broadcasted_iota(jnp.int32, logits.shape, 1)
    m1 = jnp.max(logits, axis=1, keepdims=True)
    a1 = jnp.min(jnp.where(logits == m1, iota, E + 1), axis=1, keepdims=True)
    l2 = jnp.where(iota == a1, -jnp.inf, logits)
    m2 = jnp.max(l2, axis=1, keepdims=True)
    a2 = jnp.min(jnp.where(l2 == m2, iota, E + 1), axis=1, keepdims=True)
    w1 = 1.0 / (1.0 + jnp.exp(m2 - m1))
    wts_ref[...] = jnp.concatenate([w1, 1.0 - w1], axis=1)
    eidx_ref[...] = jnp.concatenate([a1, a2], axis=1)


_router_call = pl.pallas_call(
    _router_body,
    grid=(1,),
    in_specs=[
        pl.BlockSpec((N_TOK, D), lambda i: (0, 0)),
        pl.BlockSpec((E, D), lambda i: (0, 0)),
    ],
    out_specs=[
        pl.BlockSpec((N_TOK, 2), lambda i: (0, 0)),
        pl.BlockSpec((N_TOK, 2), lambda i: (0, 0)),
    ],
    out_shape=[
        jax.ShapeDtypeStruct((N_TOK, 2), jnp.float32),
        jax.ShapeDtypeStruct((N_TOK, 2), jnp.int32),
    ],
)


def _shared_body(x_ref, wsgu_ref, wsdn_ref, wsg_ref, shared_ref):
    x = x_ref[...]
    sgu = _dot_t(x, wsgu_ref[...])
    sg = sgu[:, :F_SHARED]
    su = sgu[:, F_SHARED:]
    sh = sg * jax.nn.sigmoid(sg) * su
    sout = _dot_t(sh, wsdn_ref[...])
    gate = jax.nn.sigmoid(_dot_t(x, wsg_ref[...]))
    shared_ref[...] = gate * sout


def _make_shared_call(half):
    off = half * (NB // 2)
    return pl.pallas_call(
        _shared_body,
        grid=(NB // 2,),
        in_specs=[
            pl.BlockSpec((TB, D), lambda b: (b + off, 0)),
            pl.BlockSpec((2 * F_SHARED, D), lambda b: (0, 0)),
            pl.BlockSpec((D, F_SHARED), lambda b: (0, 0)),
            pl.BlockSpec((1, D), lambda b: (0, 0)),
        ],
        out_specs=pl.BlockSpec((TB, D), lambda b: (b, 0)),
        out_shape=jax.ShapeDtypeStruct((N_TOK // 2, D), jnp.float32),
    )


_shared_call_a = _make_shared_call(0)
_shared_call_b = _make_shared_call(1)


def _sort_body(e_ref, lt_ref, cp_ref, s0_ref, s1_ref,
               d0_ref, d1_ref, bexp_ref):
    e = e_ref[...]
    lt = lt_ref[...]
    cp = cp_ref[...]

    dest = jnp.zeros((ROWS, COLS), jnp.float32)
    off = jnp.zeros((1, 1), jnp.float32)
    rb = (TMG * lax.broadcasted_iota(jnp.int32, (8, NBG), 1)).astype(jnp.float32)
    nleq = jnp.zeros((8, NBG), jnp.float32)
    for k in range(E):
        ohk = (e == k).astype(jnp.float32)
        within = lax.dot_general(lt, ohk, (((1,), (0,)), ((), ())),
                                 preferred_element_type=jnp.float32)
        s = within[ROWS - 1:ROWS, :]
        excl = lax.dot_general(s, cp, (((1,), (0,)), ((), ())),
                               preferred_element_type=jnp.float32)
        incl = within + excl
        dest = dest + ohk * (off + incl - 1.0)
        nleq = nleq + (off <= rb).astype(jnp.float32)
        tot = excl[0:1, COLS - 1:COLS] + s[0:1, COLS - 1:COLS]
        off = off + jnp.ceil(tot * (1.0 / TMG)) * TMG
    d0_ref[...] = lax.dot_general(dest, s0_ref[...], (((1,), (0,)), ((), ())),
                                  precision=lax.Precision.HIGHEST,
                                  preferred_element_type=jnp.float32
                                  ).astype(jnp.int32)
    d1_ref[...] = lax.dot_general(dest, s1_ref[...], (((1,), (0,)), ((), ())),
                                  precision=lax.Precision.HIGHEST,
                                  preferred_element_type=jnp.float32
                                  ).astype(jnp.int32)
    bexp_ref[...] = (nleq - 1.0).astype(jnp.int32)


_sort_call = pl.pallas_call(
    _sort_body,
    grid=(1,),
    in_specs=[
        pl.BlockSpec((ROWS, COLS), lambda i: (0, 0)),
        pl.BlockSpec((ROWS, ROWS), lambda i: (0, 0)),
        pl.BlockSpec((COLS, COLS), lambda i: (0, 0)),
        pl.BlockSpec((COLS, COLS // 2), lambda i: (0, 0)),
        pl.BlockSpec((COLS, COLS // 2), lambda i: (0, 0)),
    ],
    out_specs=[
        pl.BlockSpec((ROWS, COLS // 2), lambda i: (0, 0)),
        pl.BlockSpec((ROWS, COLS // 2), lambda i: (0, 0)),
        pl.BlockSpec((8, NBG), lambda i: (0, 0)),
    ],
    out_shape=[
        jax.ShapeDtypeStruct((ROWS, COLS // 2), jnp.int32),
        jax.ShapeDtypeStruct((ROWS, COLS // 2), jnp.int32),
        jax.ShapeDtypeStruct((8, NBG), jnp.int32),
    ],
)

_LT = np.tril(np.ones((ROWS, ROWS), np.float32))
_CP = np.tril(np.ones((COLS, COLS), np.float32), -1).T
_S0 = (np.arange(COLS)[:, None] == 2 * np.arange(COLS // 2)[None, :]
       ).astype(np.float32)
_S1 = (np.arange(COLS)[:, None] == 2 * np.arange(COLS // 2)[None, :] + 1
       ).astype(np.float32)


@functools.cache
def _sc_calls():
    mesh = plsc.VectorSubcoreMesh(core_axis_name="c", subcore_axis_name="s",
                                  num_cores=SC_NC, num_subcores=SC_NS)

    @functools.partial(
        pl.kernel,
        out_type=jax.ShapeDtypeStruct((RMAX, D), jnp.float32),
        mesh=mesh,
        scratch_types=[
            pltpu.VMEM((CH,), jnp.int32),
            pltpu.VMEM((CH,), jnp.int32),
            pltpu.VMEM((CH, D), jnp.float32),
            pltpu.SemaphoreType.DMA,
        ],
    )
    def _sc_scatter(x_hbm, d0_hbm, d1_hbm, xs_hbm, idx0_v, idx1_v, rows_v, sem):
        wid = lax.axis_index("s") * SC_NC + lax.axis_index("c")
        for c in range(NCH):
            base = wid * TPW + c * CH
            pltpu.sync_copy(d0_hbm.at[pl.ds(base, CH)], idx0_v)
            pltpu.sync_copy(d1_hbm.at[pl.ds(base, CH)], idx1_v)
            pltpu.sync_copy(x_hbm.at[pl.ds(base, CH)], rows_v)
            pltpu.async_copy(rows_v, xs_hbm.at[idx0_v], sem).wait()
            pltpu.async_copy(rows_v, xs_hbm.at[idx1_v], sem).wait()

    @functools.partial(
        pl.kernel,
        out_type=[
            jax.ShapeDtypeStruct((N_TOK, D), jnp.float32),
            jax.ShapeDtypeStruct((N_TOK, D), jnp.float32),
        ],
        mesh=mesh,
        scratch_types=[
            pltpu.VMEM((CH,), jnp.int32),
            pltpu.VMEM((CH, D), jnp.float32),
            pltpu.SemaphoreType.DMA,
        ],
    )
    def _sc_gather(ys_hbm, d0_hbm, d1_hbm, y0_hbm, y1_hbm, idx_v, rows_v, sem):
        wid = lax.axis_index("s") * SC_NC + lax.axis_index("c")
        for c in range(NCH):
            base = wid * TPW + c * CH
            pltpu.sync_copy(d0_hbm.at[pl.ds(base, CH)], idx_v)
            pltpu.async_copy(ys_hbm.at[idx_v], rows_v, sem).wait()
            pltpu.sync_copy(rows_v, y0_hbm.at[pl.ds(base, CH)])
            pltpu.sync_copy(d1_hbm.at[pl.ds(base, CH)], idx_v)
            pltpu.async_copy(ys_hbm.at[idx_v], rows_v, sem).wait()
            pltpu.sync_copy(rows_v, y1_hbm.at[pl.ds(base, CH)])

    return _sc_scatter, _sc_gather


def _group_mlp_body(bexp_ref, xs_ref, wgu_ref, wdn_ref, ys_ref):
    x = xs_ref[...]
    gu = _dot_t(x, wgu_ref[0])
    g = gu[:, :F_MOE]
    u = gu[:, F_MOE:]
    h = g * jax.nn.sigmoid(g) * u
    ys_ref[...] = _dot_t(h, wdn_ref[0])


_group_mlp_call = pl.pallas_call(
    _group_mlp_body,
    grid_spec=pltpu.PrefetchScalarGridSpec(
        num_scalar_prefetch=1,
        grid=(NBG,),
        in_specs=[
            pl.BlockSpec((TMG, D), lambda b, s: (b, 0)),
            pl.BlockSpec((1, 2 * F_MOE, D), lambda b, s: (s[b], 0, 0)),
            pl.BlockSpec((1, D, F_MOE), lambda b, s: (s[b], 0, 0)),
        ],
        out_specs=pl.BlockSpec((TMG, D), lambda b, s: (b, 0)),
    ),
    out_shape=jax.ShapeDtypeStruct((RMAX, D), jnp.float32),
)


def _combine_body(sa_ref, sb_ref, wts_ref, y0_ref, y1_ref, out_ref):
    b = pl.program_id(0)
    w = wts_ref[...]
    sh = jnp.where(b < NB // 2, sa_ref[...], sb_ref[...])
    out_ref[...] = sh + w[:, 0:1] * y0_ref[...] + w[:, 1:2] * y1_ref[...]


_combine_call = pl.pallas_call(
    _combine_body,
    grid=(NB,),
    in_specs=[
        pl.BlockSpec((TB, D), lambda b: (jnp.minimum(b, NB // 2 - 1), 0)),
        pl.BlockSpec((TB, D), lambda b: (jnp.maximum(b - NB // 2, 0), 0)),
        pl.BlockSpec((TB, 2), lambda b: (b, 0)),
        pl.BlockSpec((TB, D), lambda b: (b, 0)),
        pl.BlockSpec((TB, D), lambda b: (b, 0)),
    ],
    out_specs=pl.BlockSpec((TB, D), lambda b: (b, 0)),
    out_shape=jax.ShapeDtypeStruct((N_TOK, D), jnp.float32),
)


@jax.jit
def kernel(hidden_states, Wg, W_gu, W_dn, Ws_gu, Ws_dn, Wsg):
    bs, nt, d = hidden_states.shape
    x = hidden_states.reshape(-1, d)

    wts, eidx = _router_call(x, Wg)

    e_rows = eidx.reshape(ROWS, COLS)
    d0x, d1x, bexp = _sort_call(e_rows, _LT, _CP, _S0, _S1)
    d0 = d0x.reshape(N_TOK)
    d1 = d1x.reshape(N_TOK)

    sc_scatter, sc_gather = _sc_calls()
    xs = sc_scatter(x, d0, d1)
    shared_a = _shared_call_a(x, Ws_gu, Ws_dn, Wsg)
    ys = _group_mlp_call(bexp[0], xs, W_gu, W_dn)
    shared_b = _shared_call_b(x, Ws_gu, Ws_dn, Wsg)
    y0, y1 = sc_gather(ys, d0, d1)

    out = _combine_call(shared_a, shared_b, wts, y0, y1)
    return out.reshape(bs, nt, d)
